# baseline (speedup 1.0000x reference)
import jax
import jax.numpy as jnp
from jax import lax
from jax.experimental import pallas as pl
from jax.experimental.pallas import tpu as pltpu
from jax.experimental.pallas import tpu_sc as plsc

RX = 2048
RY = 2048
F = 4
H = 64
B = 524288
FD = 8

NC = 2
NS = 16
NW = NC * NS
PER_W = B // NW
CH = 1024
GK = 128
NG = CH // GK
NPAIR = PER_W // (2 * CH)


def _sc_gather_body(idf_hbm, u_hbm, v_hbm, table_hbm, out_hbm, *scr):
    bufs = (scr[0:9], scr[9:18])
    sems = (scr[18], scr[19])
    wid = lax.axis_index("s") * NC + lax.axis_index("c")
    iot = lax.iota(jnp.int32, 16)
    rowpat = lax.shift_right_logical(iot, 2)
    colpat = lax.bitwise_and(iot, 3)
    zeros16 = jnp.zeros((16,), jnp.float32)

    for b in range(2):
        ovb = bufs[b][8]

        def clear_body(i, carry, ovb=ovb):
            ovb[pl.ds(i * 16, 16)] = zeros16
            return carry

        lax.fori_loop(0, CH * FD // 16, clear_body, 0)

    def stage(base, buf, sem):
        uu, vv, wx, wy, px0, px1, idx, rows, ov = buf
        pltpu.sync_copy(u_hbm.at[pl.ds(base, CH)], uu)
        pltpu.sync_copy(v_hbm.at[pl.ds(base, CH)], vv)

        def idx_body(i, carry2):
            s = pl.ds(i * 16, 16)
            ux = uu[s] * jnp.float32(RX)
            vy = vv[s] * jnp.float32(RY)
            x0 = ux.astype(jnp.int32)
            y0 = vy.astype(jnp.int32)
            x0 = jnp.where(x0 == RX, 0, x0)
            y0 = jnp.minimum(y0, RY - 1)
            x1 = jnp.where(x0 + 1 == RX, RX - 1, x0 + 1)
            y1 = jnp.minimum(y0 + 1, RY - 1)
            wx[s] = ux - x0.astype(jnp.float32)
            wy[s] = vy - y0.astype(jnp.float32)
            px0[s] = lax.bitwise_and(x0, 1) * F
            px1[s] = lax.bitwise_and(x1, 1) * F
            row0 = y0 * RX
            row1 = y1 * RX
            j = lax.shift_right_logical(jnp.int32(i * 16), jnp.int32(7))
            k = jnp.int32(i * 16) - j * GK
            sk = pl.ds(k, 16)
            idx[0, j, sk] = lax.shift_right_logical(row0 + x0, 1)
            idx[1, j, sk] = lax.shift_right_logical(row0 + x1, 1)
            idx[2, j, sk] = lax.shift_right_logical(row1 + x0, 1)
            idx[3, j, sk] = lax.shift_right_logical(row1 + x1, 1)
            return carry2

        lax.fori_loop(0, CH // 16, idx_body, 0)

        copies = []
        for q in range(4):
            for j in range(NG):
                copies.append(pltpu.async_copy(
                    table_hbm.at[idx.at[q, j]], rows.at[q, j], sem))
        return copies

    def finish(base, buf, copies):
        uu, vv, wx, wy, px0, px1, idx, rows, ov = buf
        for cp in copies:
            cp.wait()

        def interp_body(i, carry2):
            p = i * 4 + rowpat
            gj = lax.shift_right_logical(p, jnp.int32(7))
            gk = lax.bitwise_and(p, jnp.int32(GK - 1))
            o0 = plsc.load_gather(px0, [p]) + colpat
            o1 = plsc.load_gather(px1, [p]) + colpat
            v00 = plsc.load_gather(rows.at[0], [gj, gk, o0])
            v10 = plsc.load_gather(rows.at[1], [gj, gk, o1])
            v01 = plsc.load_gather(rows.at[2], [gj, gk, o0])
            v11 = plsc.load_gather(rows.at[3], [gj, gk, o1])
            wxv = plsc.load_gather(wx, [p])
            wyv = plsc.load_gather(wy, [p])
            vup = v00 + wxv * (v10 - v00)
            vdn = v01 + wxv * (v11 - v01)
            res = vup + wyv * (vdn - vup)
            plsc.store_scatter(ov, [p * FD + colpat + 1], res)
            return carry2

        lax.fori_loop(0, CH // 4, interp_body, 0)

        pltpu.sync_copy(idf_hbm.at[pl.ds(base, CH)], uu)

        def idf_body(i, carry2):
            p = i * 16 + iot
            plsc.store_scatter(ov, [p * FD], uu[pl.ds(i * 16, 16)])
            return carry2

        lax.fori_loop(0, CH // 16, idf_body, 0)

        pltpu.sync_copy(ov, out_hbm.at[pl.ds(base * FD, CH * FD)])

    def pair_body(t, carry):
        base_a = wid * PER_W + (2 * t) * CH
        base_b = base_a + CH
        ca = stage(base_a, bufs[0], sems[0])
        cb = stage(base_b, bufs[1], sems[1])
        finish(base_a, bufs[0], ca)
        finish(base_b, bufs[1], cb)
        return carry

    lax.fori_loop(0, NPAIR, pair_body, 0)


def _sc_gather(idf, u, v, table):
    mesh = plsc.VectorSubcoreMesh(core_axis_name="c", subcore_axis_name="s",
                                  num_cores=NC, num_subcores=NS)
    buf_types = [
        pltpu.VMEM((CH,), jnp.float32),
        pltpu.VMEM((CH,), jnp.float32),
        pltpu.VMEM((CH,), jnp.float32),
        pltpu.VMEM((CH,), jnp.float32),
        pltpu.VMEM((CH,), jnp.int32),
        pltpu.VMEM((CH,), jnp.int32),
        pltpu.VMEM((4, NG, GK), jnp.int32),
        pltpu.VMEM((4, NG, GK, 2 * F), jnp.float32),
        pltpu.VMEM((CH * FD,), jnp.float32),
    ]
    f = pl.kernel(
        _sc_gather_body,
        out_type=jax.ShapeDtypeStruct((B * FD,), jnp.float32),
        mesh=mesh,
        compiler_params=pltpu.CompilerParams(needs_layout_passes=False,
                                             use_tc_tiling_on_sc=False),
        scratch_types=buf_types + buf_types + [
            pltpu.SemaphoreType.DMA,
            pltpu.SemaphoreType.DMA,
        ],
    )
    tpair = jnp.stack([table[0::2, 0], table[0::2, 1],
                       table[0::2, 2], table[0::2, 3],
                       table[1::2, 0], table[1::2, 1],
                       table[1::2, 2], table[1::2, 3]], axis=1)
    return f(idf, u, v, tpair)


import numpy as np

_SCATTER = np.zeros((F, 128, 512), np.float32)
for _f in range(F):
    for _q in range(128):
        _SCATTER[_f, _q, 4 * _q + _f] = 1.0


def _to_row_major(table):
    R = RX * RY // 128
    tT3 = table.T.reshape(F, R, 128)
    return lax.dot_general(tT3, jnp.asarray(_SCATTER),
                           (((0, 2), (0, 1)), ((), ())),
                           precision=lax.Precision.HIGH)


BT = 2048


def _mlp_body(f_ref, w0_ref, b0_ref, w1_ref, b1_ref, w2_ref, b2_ref, o_ref):
    feat = f_ref[...].astype(jnp.bfloat16)
    h = jnp.dot(feat, w0_ref[...], preferred_element_type=jnp.float32)
    h = jnp.maximum(h + b0_ref[...], 0.0).astype(jnp.bfloat16)
    h2 = jnp.dot(h, w1_ref[...], preferred_element_type=jnp.float32)
    h2 = jnp.maximum(h2 + b1_ref[...], 0.0).astype(jnp.bfloat16)
    o = lax.dot_general(w2_ref[...], h2, (((1,), (1,)), ((), ())),
                        preferred_element_type=jnp.float32)
    o_ref[...] = jax.nn.sigmoid(o + b2_ref[...])


def _mlp(featw, w0p, b0, w1t, b1, w2p4, b2p4):
    grid = (B // BT,)
    return pl.pallas_call(
        _mlp_body,
        grid=grid,
        in_specs=[
            pl.BlockSpec((BT, FD), lambda i: (i, 0)),
            pl.BlockSpec((FD, H), lambda i: (0, 0)),
            pl.BlockSpec((1, H), lambda i: (0, 0)),
            pl.BlockSpec((H, H), lambda i: (0, 0)),
            pl.BlockSpec((1, H), lambda i: (0, 0)),
            pl.BlockSpec((F, H), lambda i: (0, 0)),
            pl.BlockSpec((F, 1), lambda i: (0, 0)),
        ],
        out_specs=pl.BlockSpec((F, BT), lambda i: (0, i)),
        out_shape=jax.ShapeDtypeStruct((F, B), jnp.float32),
    )(featw, w0p, b0, w1t, b1, w2p4, b2p4)


def kernel(x, table, W0, b0, W1, b1, W2, b2):
    idf = x[:, 0]
    u = x[:, 1]
    v = x[:, 2]
    featw = _sc_gather(idf, u, v, table).reshape(B, FD)
    w0p = jnp.pad(W0.T, ((0, FD - 1 - F), (0, 0))).astype(jnp.bfloat16)
    w2p4 = jnp.pad(W2, ((0, 1), (0, 0))).astype(jnp.bfloat16)
    b2p4 = jnp.pad(b2, (0, 1)).reshape(F, 1)
    out4 = _mlp(featw, w0p, b0.reshape(1, H), W1.T.astype(jnp.bfloat16),
                b1.reshape(1, H), w2p4, b2p4)
    return out4[:3].T


# speedup vs baseline: 8.1765x; 8.1765x over previous
import jax
import jax.numpy as jnp
from jax import lax
from jax.experimental import pallas as pl
from jax.experimental.pallas import tpu as pltpu
from jax.experimental.pallas import tpu_sc as plsc

RX = 2048
RY = 2048
F = 4
H = 64
B = 524288
FD = 8

NC = 2
NS = 16
NW = NC * NS
PER_W = B // NW
CH = 1024
GK = 128
NG = CH // GK
NPAIR = PER_W // (2 * CH)


def _sc_gather_body(idf_hbm, u_hbm, v_hbm, table_hbm, out_hbm, *scr):
    bufs = (scr[0:9], scr[9:18])
    sems = (scr[18], scr[19])
    wid = lax.axis_index("s") * NC + lax.axis_index("c")
    iot = lax.iota(jnp.int32, 16)
    rowpat = lax.shift_right_logical(iot, 2)
    colpat = lax.bitwise_and(iot, 3)
    zeros16 = jnp.zeros((16,), jnp.float32)

    for b in range(2):
        ovb = bufs[b][8]

        def clear_body(i, carry, ovb=ovb):
            ovb[pl.ds(i * 16, 16)] = zeros16
            return carry

        lax.fori_loop(0, CH * FD // 16, clear_body, 0)

    def stage(base, buf, sem):
        uu, vv, wx, wy, px0, px1, idx, rows, ov = buf
        pltpu.sync_copy(u_hbm.at[pl.ds(base, CH)], uu)
        pltpu.sync_copy(v_hbm.at[pl.ds(base, CH)], vv)

        def idx_body(i, carry2):
            s = pl.ds(i * 16, 16)
            ux = uu[s] * jnp.float32(RX)
            vy = vv[s] * jnp.float32(RY)
            x0 = ux.astype(jnp.int32)
            y0 = vy.astype(jnp.int32)
            x0 = jnp.where(x0 == RX, 0, x0)
            y0 = jnp.minimum(y0, RY - 1)
            x1 = jnp.where(x0 + 1 == RX, RX - 1, x0 + 1)
            y1 = jnp.minimum(y0 + 1, RY - 1)
            wx[s] = ux - x0.astype(jnp.float32)
            wy[s] = vy - y0.astype(jnp.float32)
            px0[s] = lax.bitwise_and(x0, 1) * F
            px1[s] = lax.bitwise_and(x1, 1) * F
            row0 = y0 * RX
            row1 = y1 * RX
            j = lax.shift_right_logical(jnp.int32(i * 16), jnp.int32(7))
            k = jnp.int32(i * 16) - j * GK
            sk = pl.ds(k, 16)
            idx[0, j, sk] = lax.shift_right_logical(row0 + x0, 1)
            idx[1, j, sk] = lax.shift_right_logical(row0 + x1, 1)
            idx[2, j, sk] = lax.shift_right_logical(row1 + x0, 1)
            idx[3, j, sk] = lax.shift_right_logical(row1 + x1, 1)
            return carry2

        lax.fori_loop(0, CH // 16, idx_body, 0)

        copies = []
        for q in range(4):
            for j in range(NG):
                copies.append(pltpu.async_copy(
                    table_hbm.at[idx.at[q, j]], rows.at[q, j], sem))
        return copies

    def finish(base, buf, copies):
        uu, vv, wx, wy, px0, px1, idx, rows, ov = buf
        for cp in copies:
            cp.wait()

        def interp_body(i, carry2):
            p = i * 4 + rowpat
            gj = lax.shift_right_logical(p, jnp.int32(7))
            gk = lax.bitwise_and(p, jnp.int32(GK - 1))
            o0 = plsc.load_gather(px0, [p]) + colpat
            o1 = plsc.load_gather(px1, [p]) + colpat
            v00 = plsc.load_gather(rows.at[0], [gj, gk, o0])
            v10 = plsc.load_gather(rows.at[1], [gj, gk, o1])
            v01 = plsc.load_gather(rows.at[2], [gj, gk, o0])
            v11 = plsc.load_gather(rows.at[3], [gj, gk, o1])
            wxv = plsc.load_gather(wx, [p])
            wyv = plsc.load_gather(wy, [p])
            vup = v00 + wxv * (v10 - v00)
            vdn = v01 + wxv * (v11 - v01)
            res = vup + wyv * (vdn - vup)
            plsc.store_scatter(ov, [p * FD + colpat + 1], res)
            return carry2

        lax.fori_loop(0, CH // 4, interp_body, 0)

        pltpu.sync_copy(idf_hbm.at[pl.ds(base, CH)], uu)

        def idf_body(i, carry2):
            p = i * 16 + iot
            plsc.store_scatter(ov, [p * FD], uu[pl.ds(i * 16, 16)])
            return carry2

        lax.fori_loop(0, CH // 16, idf_body, 0)

        pltpu.sync_copy(ov, out_hbm.at[pl.ds(base * FD, CH * FD)])

    def pair_body(t, carry):
        base_a = wid * PER_W + (2 * t) * CH
        base_b = base_a + CH
        ca = stage(base_a, bufs[0], sems[0])
        cb = stage(base_b, bufs[1], sems[1])
        finish(base_a, bufs[0], ca)
        finish(base_b, bufs[1], cb)
        return carry

    lax.fori_loop(0, NPAIR, pair_body, 0)


def _sc_gather(idf, u, v, table):
    mesh = plsc.VectorSubcoreMesh(core_axis_name="c", subcore_axis_name="s",
                                  num_cores=NC, num_subcores=NS)
    buf_types = [
        pltpu.VMEM((CH,), jnp.float32),
        pltpu.VMEM((CH,), jnp.float32),
        pltpu.VMEM((CH,), jnp.float32),
        pltpu.VMEM((CH,), jnp.float32),
        pltpu.VMEM((CH,), jnp.int32),
        pltpu.VMEM((CH,), jnp.int32),
        pltpu.VMEM((4, NG, GK), jnp.int32),
        pltpu.VMEM((4, NG, GK, 2 * F), jnp.float32),
        pltpu.VMEM((CH * FD,), jnp.float32),
    ]
    f = pl.kernel(
        _sc_gather_body,
        out_type=jax.ShapeDtypeStruct((B * FD,), jnp.float32),
        mesh=mesh,
        compiler_params=pltpu.CompilerParams(needs_layout_passes=False,
                                             use_tc_tiling_on_sc=False),
        scratch_types=buf_types + buf_types + [
            pltpu.SemaphoreType.DMA,
            pltpu.SemaphoreType.DMA,
        ],
    )
    t128 = _to_row_major(table)
    return f(idf, u, v, t128.reshape(RX * RY // 2, 2 * F))


import numpy as np

_SCATTER = np.zeros((F, 128, 512), np.float32)
for _f in range(F):
    for _q in range(128):
        _SCATTER[_f, _q, 4 * _q + _f] = 1.0


def _to_row_major(table):
    R = RX * RY // 128
    tT3 = table.T.reshape(F, R, 128)
    return lax.dot_general(tT3, jnp.asarray(_SCATTER),
                           (((0, 2), (0, 1)), ((), ())),
                           precision=lax.Precision.DEFAULT)


BT = 8192


def _mlp_body(f_ref, w0_ref, b0_ref, w1_ref, b1_ref, w2_ref, b2_ref, o_ref):
    feat = f_ref[...].astype(jnp.bfloat16)
    h = jnp.dot(feat, w0_ref[...], preferred_element_type=jnp.float32)
    h = jnp.maximum(h + b0_ref[...], 0.0).astype(jnp.bfloat16)
    h2 = jnp.dot(h, w1_ref[...], preferred_element_type=jnp.float32)
    h2 = jnp.maximum(h2 + b1_ref[...], 0.0).astype(jnp.bfloat16)
    o = lax.dot_general(w2_ref[...], h2, (((1,), (1,)), ((), ())),
                        preferred_element_type=jnp.float32)
    o_ref[...] = jax.nn.sigmoid(o + b2_ref[...])


def _mlp(featw, w0p, b0, w1t, b1, w2p4, b2p4):
    grid = (B // BT,)
    return pl.pallas_call(
        _mlp_body,
        grid=grid,
        in_specs=[
            pl.BlockSpec((BT, FD), lambda i: (i, 0)),
            pl.BlockSpec((FD, H), lambda i: (0, 0)),
            pl.BlockSpec((1, H), lambda i: (0, 0)),
            pl.BlockSpec((H, H), lambda i: (0, 0)),
            pl.BlockSpec((1, H), lambda i: (0, 0)),
            pl.BlockSpec((F, H), lambda i: (0, 0)),
            pl.BlockSpec((F, 1), lambda i: (0, 0)),
        ],
        out_specs=pl.BlockSpec((F, BT), lambda i: (0, i)),
        out_shape=jax.ShapeDtypeStruct((F, B), jnp.float32),
    )(featw, w0p, b0, w1t, b1, w2p4, b2p4)


def kernel(x, table, W0, b0, W1, b1, W2, b2):
    idf = x[:, 0]
    u = x[:, 1]
    v = x[:, 2]
    featw = _sc_gather(idf, u, v, table).reshape(B, FD)
    w0p = jnp.pad(W0.T, ((0, FD - 1 - F), (0, 0))).astype(jnp.bfloat16)
    w2p4 = jnp.pad(W2, ((0, 1), (0, 0))).astype(jnp.bfloat16)
    b2p4 = jnp.pad(b2, (0, 1)).reshape(F, 1)
    out4 = _mlp(featw, w0p, b0.reshape(1, H), W1.T.astype(jnp.bfloat16),
                b1.reshape(1, H), w2p4, b2p4)
    return out4[:3].T


# speedup vs baseline: 8.7005x; 1.0641x over previous
import jax
import jax.numpy as jnp
from jax import lax
from jax.experimental import pallas as pl
from jax.experimental.pallas import tpu as pltpu
from jax.experimental.pallas import tpu_sc as plsc

RX = 2048
RY = 2048
F = 4
H = 64
B = 524288
FD = 8

NC = 2
NS = 16
NW = NC * NS
B2 = B // 2
PER_W = B2 // NW
CH = 1024
GK = 128
NG = CH // GK
NPAIR = PER_W // (2 * CH)


def _sc_gather_body(idf_hbm, u_hbm, v_hbm, table_hbm, out_hbm, *scr):
    bufs = (scr[0:9], scr[9:18])
    sems = (scr[18], scr[19])
    wid = lax.axis_index("s") * NC + lax.axis_index("c")
    iot = lax.iota(jnp.int32, 16)
    rowpat = lax.shift_right_logical(iot, 2)
    colpat = lax.bitwise_and(iot, 3)
    zeros16 = jnp.zeros((16,), jnp.float32)

    for b in range(2):
        ovb = bufs[b][8]

        def clear_body(i, carry, ovb=ovb):
            ovb[pl.ds(i * 16, 16)] = zeros16
            return carry

        lax.fori_loop(0, CH * FD // 16, clear_body, 0)

    def stage(base, buf, sem):
        uu, vv, wx, wy, px0, px1, idx, rows, ov = buf
        pltpu.sync_copy(u_hbm.at[pl.ds(base, CH)], uu)
        pltpu.sync_copy(v_hbm.at[pl.ds(base, CH)], vv)

        def idx_body(i, carry2):
            s = pl.ds(i * 16, 16)
            ux = uu[s] * jnp.float32(RX)
            vy = vv[s] * jnp.float32(RY)
            x0 = ux.astype(jnp.int32)
            y0 = vy.astype(jnp.int32)
            x0 = jnp.where(x0 == RX, 0, x0)
            y0 = jnp.minimum(y0, RY - 1)
            x1 = jnp.where(x0 + 1 == RX, RX - 1, x0 + 1)
            y1 = jnp.minimum(y0 + 1, RY - 1)
            wx[s] = ux - x0.astype(jnp.float32)
            wy[s] = vy - y0.astype(jnp.float32)
            px0[s] = lax.bitwise_and(x0, 1) * F
            px1[s] = lax.bitwise_and(x1, 1) * F
            row0 = y0 * RX
            row1 = y1 * RX
            j = lax.shift_right_logical(jnp.int32(i * 16), jnp.int32(7))
            k = jnp.int32(i * 16) - j * GK
            sk = pl.ds(k, 16)
            idx[0, j, sk] = lax.shift_right_logical(row0 + x0, 1)
            idx[1, j, sk] = lax.shift_right_logical(row0 + x1, 1)
            idx[2, j, sk] = lax.shift_right_logical(row1 + x0, 1)
            idx[3, j, sk] = lax.shift_right_logical(row1 + x1, 1)
            return carry2

        lax.fori_loop(0, CH // 16, idx_body, 0)

        copies = []
        for q in range(4):
            for j in range(NG):
                copies.append(pltpu.async_copy(
                    table_hbm.at[idx.at[q, j]], rows.at[q, j], sem))
        return copies

    def finish(base, buf, copies):
        uu, vv, wx, wy, px0, px1, idx, rows, ov = buf
        for cp in copies:
            cp.wait()

        def interp_body(i, carry2):
            p = i * 4 + rowpat
            gj = lax.shift_right_logical(p, jnp.int32(7))
            gk = lax.bitwise_and(p, jnp.int32(GK - 1))
            o0 = plsc.load_gather(px0, [p]) + colpat
            o1 = plsc.load_gather(px1, [p]) + colpat
            v00 = plsc.load_gather(rows.at[0], [gj, gk, o0])
            v10 = plsc.load_gather(rows.at[1], [gj, gk, o1])
            v01 = plsc.load_gather(rows.at[2], [gj, gk, o0])
            v11 = plsc.load_gather(rows.at[3], [gj, gk, o1])
            wxv = plsc.load_gather(wx, [p])
            wyv = plsc.load_gather(wy, [p])
            vup = v00 + wxv * (v10 - v00)
            vdn = v01 + wxv * (v11 - v01)
            res = vup + wyv * (vdn - vup)
            plsc.store_scatter(ov, [p * FD + colpat + 1], res)
            return carry2

        lax.fori_loop(0, CH // 4, interp_body, 0)

        pltpu.sync_copy(idf_hbm.at[pl.ds(base, CH)], uu)

        def idf_body(i, carry2):
            p = i * 16 + iot
            plsc.store_scatter(ov, [p * FD], uu[pl.ds(i * 16, 16)])
            return carry2

        lax.fori_loop(0, CH // 16, idf_body, 0)

        pltpu.sync_copy(ov, out_hbm.at[pl.ds(base * FD, CH * FD)])

    def pair_body(t, carry):
        base_a = wid * PER_W + (2 * t) * CH
        base_b = base_a + CH
        ca = stage(base_a, bufs[0], sems[0])
        cb = stage(base_b, bufs[1], sems[1])
        finish(base_a, bufs[0], ca)
        finish(base_b, bufs[1], cb)
        return carry

    lax.fori_loop(0, NPAIR, pair_body, 0)


def _sc_gather(idf, u, v, table):
    mesh = plsc.VectorSubcoreMesh(core_axis_name="c", subcore_axis_name="s",
                                  num_cores=NC, num_subcores=NS)
    buf_types = [
        pltpu.VMEM((CH,), jnp.float32),
        pltpu.VMEM((CH,), jnp.float32),
        pltpu.VMEM((CH,), jnp.float32),
        pltpu.VMEM((CH,), jnp.float32),
        pltpu.VMEM((CH,), jnp.int32),
        pltpu.VMEM((CH,), jnp.int32),
        pltpu.VMEM((4, NG, GK), jnp.int32),
        pltpu.VMEM((4, NG, GK, 2 * F), jnp.float32),
        pltpu.VMEM((CH * FD,), jnp.float32),
    ]
    f = pl.kernel(
        _sc_gather_body,
        out_type=jax.ShapeDtypeStruct((B2 * FD,), jnp.float32),
        mesh=mesh,
        compiler_params=pltpu.CompilerParams(needs_layout_passes=False,
                                             use_tc_tiling_on_sc=False),
        scratch_types=buf_types + buf_types + [
            pltpu.SemaphoreType.DMA,
            pltpu.SemaphoreType.DMA,
        ],
    )
    return f(idf, u, v, table)


import numpy as np

_SCATTER = np.zeros((F, 128, 512), np.float32)
for _f in range(F):
    for _q in range(128):
        _SCATTER[_f, _q, 4 * _q + _f] = 1.0


def _to_row_major(table):
    R = RX * RY // 128
    tT3 = table.T.reshape(F, R, 128)
    return lax.dot_general(tT3, jnp.asarray(_SCATTER),
                           (((0, 2), (0, 1)), ((), ())),
                           precision=lax.Precision.DEFAULT)


BT = 8192


def _mlp_body(f_ref, w0_ref, b0_ref, w1_ref, b1_ref, w2_ref, b2_ref, o_ref):
    feat = f_ref[...].astype(jnp.bfloat16)
    h = jnp.dot(feat, w0_ref[...], preferred_element_type=jnp.float32)
    h = jnp.maximum(h + b0_ref[...], 0.0).astype(jnp.bfloat16)
    h2 = jnp.dot(h, w1_ref[...], preferred_element_type=jnp.float32)
    h2 = jnp.maximum(h2 + b1_ref[...], 0.0).astype(jnp.bfloat16)
    o = lax.dot_general(w2_ref[...], h2, (((1,), (1,)), ((), ())),
                        preferred_element_type=jnp.float32)
    o_ref[...] = jax.nn.sigmoid(o + b2_ref[...])


def _mlp(featw, w0p, b0, w1t, b1, w2p4, b2p4):
    grid = (B2 // BT,)
    return pl.pallas_call(
        _mlp_body,
        grid=grid,
        in_specs=[
            pl.BlockSpec((BT, FD), lambda i: (i, 0)),
            pl.BlockSpec((FD, H), lambda i: (0, 0)),
            pl.BlockSpec((1, H), lambda i: (0, 0)),
            pl.BlockSpec((H, H), lambda i: (0, 0)),
            pl.BlockSpec((1, H), lambda i: (0, 0)),
            pl.BlockSpec((F, H), lambda i: (0, 0)),
            pl.BlockSpec((F, 1), lambda i: (0, 0)),
        ],
        out_specs=pl.BlockSpec((F, BT), lambda i: (0, i)),
        out_shape=jax.ShapeDtypeStruct((F, B2), jnp.float32),
    )(featw, w0p, b0, w1t, b1, w2p4, b2p4)


def kernel(x, table, W0, b0, W1, b1, W2, b2):
    idf = x[:, 0]
    u = x[:, 1]
    v = x[:, 2]
    t8 = _to_row_major(table).reshape(RX * RY // 2, 2 * F)
    w0p = jnp.pad(W0.T, ((0, FD - 1 - F), (0, 0))).astype(jnp.bfloat16)
    w2p4 = jnp.pad(W2, ((0, 1), (0, 0))).astype(jnp.bfloat16)
    b2p4 = jnp.pad(b2, (0, 1)).reshape(F, 1)
    w1t = W1.T.astype(jnp.bfloat16)
    b0r = b0.reshape(1, H)
    b1r = b1.reshape(1, H)
    outs = []
    for hstart in (0, B2):
        sl = slice(hstart, hstart + B2)
        featw = _sc_gather(idf[sl], u[sl], v[sl], t8).reshape(B2, FD)
        outs.append(_mlp(featw, w0p, b0r, w1t, b1r, w2p4, b2p4))
    return jnp.concatenate([outs[0][:3].T, outs[1][:3].T], axis=0)


# speedup vs baseline: 13.3225x; 1.5312x over previous
import jax
import jax.numpy as jnp
from jax import lax
from jax.experimental import pallas as pl
from jax.experimental.pallas import tpu as pltpu
from jax.experimental.pallas import tpu_sc as plsc

RX = 2048
RY = 2048
F = 4
H = 64
B = 524288
FD = 8

NC = 2
NS = 16
NW = NC * NS
B2 = B // 4
PER_W = B2 // NW
CH = 1024
GK = 128
NG = CH // GK
NPAIR = PER_W // (2 * CH)


def _sc_gather_body(idf_hbm, u_hbm, v_hbm, table_hbm, out_hbm, *scr):
    bufs = (scr[0:9], scr[9:18])
    sems = (scr[18], scr[19])
    wid = lax.axis_index("s") * NC + lax.axis_index("c")
    iot = lax.iota(jnp.int32, 16)
    rowpat = lax.shift_right_logical(iot, 2)
    colpat = lax.bitwise_and(iot, 3)
    zeros16 = jnp.zeros((16,), jnp.float32)

    for b in range(2):
        ovb = bufs[b][8]

        def clear_body(i, carry, ovb=ovb):
            ovb[pl.ds(i * 16, 16)] = zeros16
            return carry

        lax.fori_loop(0, CH * FD // 16, clear_body, 0)

    def stage(base, buf, sem):
        uu, vv, wx, wy, px0, px1, idx, rows, ov = buf
        pltpu.sync_copy(u_hbm.at[pl.ds(base, CH)], uu)
        pltpu.sync_copy(v_hbm.at[pl.ds(base, CH)], vv)

        def idx_body(i, carry2):
            s = pl.ds(i * 16, 16)
            ux = uu[s] * jnp.float32(RX)
            vy = vv[s] * jnp.float32(RY)
            x0 = ux.astype(jnp.int32)
            y0 = vy.astype(jnp.int32)
            x0 = jnp.where(x0 == RX, 0, x0)
            y0 = jnp.minimum(y0, RY - 1)
            x1 = jnp.where(x0 + 1 == RX, RX - 1, x0 + 1)
            y1 = jnp.minimum(y0 + 1, RY - 1)
            wx[s] = ux - x0.astype(jnp.float32)
            wy[s] = vy - y0.astype(jnp.float32)
            px0[s] = lax.bitwise_and(x0, 1) * F
            px1[s] = lax.bitwise_and(x1, 1) * F
            row0 = y0 * RX
            row1 = y1 * RX
            j = lax.shift_right_logical(jnp.int32(i * 16), jnp.int32(7))
            k = jnp.int32(i * 16) - j * GK
            sk = pl.ds(k, 16)
            idx[0, j, sk] = lax.shift_right_logical(row0 + x0, 1)
            idx[1, j, sk] = lax.shift_right_logical(row0 + x1, 1)
            idx[2, j, sk] = lax.shift_right_logical(row1 + x0, 1)
            idx[3, j, sk] = lax.shift_right_logical(row1 + x1, 1)
            return carry2

        lax.fori_loop(0, CH // 16, idx_body, 0)

        copies = []
        for q in range(4):
            for j in range(NG):
                copies.append(pltpu.async_copy(
                    table_hbm.at[idx.at[q, j]], rows.at[q, j], sem))
        return copies

    def finish(base, buf, copies):
        uu, vv, wx, wy, px0, px1, idx, rows, ov = buf
        for cp in copies:
            cp.wait()

        def interp_body(i, carry2):
            p = i * 4 + rowpat
            gj = lax.shift_right_logical(p, jnp.int32(7))
            gk = lax.bitwise_and(p, jnp.int32(GK - 1))
            o0 = plsc.load_gather(px0, [p]) + colpat
            o1 = plsc.load_gather(px1, [p]) + colpat
            v00 = plsc.load_gather(rows.at[0], [gj, gk, o0])
            v10 = plsc.load_gather(rows.at[1], [gj, gk, o1])
            v01 = plsc.load_gather(rows.at[2], [gj, gk, o0])
            v11 = plsc.load_gather(rows.at[3], [gj, gk, o1])
            wxv = plsc.load_gather(wx, [p])
            wyv = plsc.load_gather(wy, [p])
            vup = v00 + wxv * (v10 - v00)
            vdn = v01 + wxv * (v11 - v01)
            res = vup + wyv * (vdn - vup)
            plsc.store_scatter(ov, [p * FD + colpat + 1], res)
            return carry2

        lax.fori_loop(0, CH // 4, interp_body, 0)

        pltpu.sync_copy(idf_hbm.at[pl.ds(base, CH)], uu)

        def idf_body(i, carry2):
            p = i * 16 + iot
            plsc.store_scatter(ov, [p * FD], uu[pl.ds(i * 16, 16)])
            return carry2

        lax.fori_loop(0, CH // 16, idf_body, 0)

        pltpu.sync_copy(ov, out_hbm.at[pl.ds(base * FD, CH * FD)])

    def pair_body(t, carry):
        base_a = wid * PER_W + (2 * t) * CH
        base_b = base_a + CH
        ca = stage(base_a, bufs[0], sems[0])
        cb = stage(base_b, bufs[1], sems[1])
        finish(base_a, bufs[0], ca)
        finish(base_b, bufs[1], cb)
        return carry

    lax.fori_loop(0, NPAIR, pair_body, 0)


def _sc_gather(idf, u, v, table):
    mesh = plsc.VectorSubcoreMesh(core_axis_name="c", subcore_axis_name="s",
                                  num_cores=NC, num_subcores=NS)
    buf_types = [
        pltpu.VMEM((CH,), jnp.float32),
        pltpu.VMEM((CH,), jnp.float32),
        pltpu.VMEM((CH,), jnp.float32),
        pltpu.VMEM((CH,), jnp.float32),
        pltpu.VMEM((CH,), jnp.int32),
        pltpu.VMEM((CH,), jnp.int32),
        pltpu.VMEM((4, NG, GK), jnp.int32),
        pltpu.VMEM((4, NG, GK, 2 * F), jnp.float32),
        pltpu.VMEM((CH * FD,), jnp.float32),
    ]
    f = pl.kernel(
        _sc_gather_body,
        out_type=jax.ShapeDtypeStruct((B2 * FD,), jnp.float32),
        mesh=mesh,
        compiler_params=pltpu.CompilerParams(needs_layout_passes=False,
                                             use_tc_tiling_on_sc=False),
        scratch_types=buf_types + buf_types + [
            pltpu.SemaphoreType.DMA,
            pltpu.SemaphoreType.DMA,
        ],
    )
    return f(idf, u, v, table)


import numpy as np

_SCATTER = np.zeros((F, 128, 512), np.float32)
for _f in range(F):
    for _q in range(128):
        _SCATTER[_f, _q, 4 * _q + _f] = 1.0


def _to_row_major(table):
    R = RX * RY // 128
    tT3 = table.T.reshape(F, R, 128)
    return lax.dot_general(tT3, jnp.asarray(_SCATTER),
                           (((0, 2), (0, 1)), ((), ())),
                           precision=lax.Precision.DEFAULT)


BT = 8192


def _mlp_body(f_ref, w0_ref, b0_ref, w1_ref, b1_ref, w2_ref, b2_ref, o_ref):
    feat = f_ref[...].astype(jnp.bfloat16)
    h = jnp.dot(feat, w0_ref[...], preferred_element_type=jnp.float32)
    h = jnp.maximum(h + b0_ref[...], 0.0).astype(jnp.bfloat16)
    h2 = jnp.dot(h, w1_ref[...], preferred_element_type=jnp.float32)
    h2 = jnp.maximum(h2 + b1_ref[...], 0.0).astype(jnp.bfloat16)
    o = lax.dot_general(w2_ref[...], h2, (((1,), (1,)), ((), ())),
                        preferred_element_type=jnp.float32)
    o_ref[...] = jax.nn.sigmoid(o + b2_ref[...])


def _mlp(featw, w0p, b0, w1t, b1, w2p4, b2p4):
    grid = (B2 // BT,)
    return pl.pallas_call(
        _mlp_body,
        grid=grid,
        in_specs=[
            pl.BlockSpec((BT, FD), lambda i: (i, 0)),
            pl.BlockSpec((FD, H), lambda i: (0, 0)),
            pl.BlockSpec((1, H), lambda i: (0, 0)),
            pl.BlockSpec((H, H), lambda i: (0, 0)),
            pl.BlockSpec((1, H), lambda i: (0, 0)),
            pl.BlockSpec((F, H), lambda i: (0, 0)),
            pl.BlockSpec((F, 1), lambda i: (0, 0)),
        ],
        out_specs=pl.BlockSpec((F, BT), lambda i: (0, i)),
        out_shape=jax.ShapeDtypeStruct((F, B2), jnp.float32),
    )(featw, w0p, b0, w1t, b1, w2p4, b2p4)


def kernel(x, table, W0, b0, W1, b1, W2, b2):
    idf = x[:, 0]
    u = x[:, 1]
    v = x[:, 2]
    t8 = _to_row_major(table).reshape(RX * RY // 2, 2 * F)
    w0p = jnp.pad(W0.T, ((0, FD - 1 - F), (0, 0))).astype(jnp.bfloat16)
    w2p4 = jnp.pad(W2, ((0, 1), (0, 0))).astype(jnp.bfloat16)
    b2p4 = jnp.pad(b2, (0, 1)).reshape(F, 1)
    w1t = W1.T.astype(jnp.bfloat16)
    b0r = b0.reshape(1, H)
    b1r = b1.reshape(1, H)
    outs = []
    for hstart in (0, B2):
        sl = slice(hstart, hstart + B2)
        featw = _sc_gather(idf[sl], u[sl], v[sl], t8).reshape(B2, FD)
        outs.append(_mlp(featw, w0p, b0r, w1t, b1r, w2p4, b2p4))
    return jnp.concatenate([outs[0][:3].T, outs[1][:3].T], axis=0)
